# baseline (speedup 1.0000x reference)
import functools

import jax
import jax.numpy as jnp
from jax import lax
from jax.experimental import pallas as pl
from jax.experimental.pallas import tpu as pltpu
from jax.experimental.pallas import tpu_sc as plsc

VOXEL_SHAPE = (4, 200, 200)
V = VOXEL_SHAPE[0] * VOXEL_SHAPE[1] * VOXEL_SHAPE[2]
NCAM = 6
C = 32
D, H, W = 32, 32, 60
DHW = D * H * W

NC, NS, L = 2, 16, 16
NW = NC * NS
CH = 512
CPW = 10
STRIDE = V // NW
NBUF = 2
GROUPS = CH // L
IGNORE = -1

_mesh = plsc.VectorSubcoreMesh(
    core_axis_name="c", subcore_axis_name="s", num_cores=NC, num_subcores=NS
)


@functools.partial(
    pl.kernel,
    out_type=jax.ShapeDtypeStruct((C, V), jnp.float32),
    mesh=_mesh,
    scratch_types=(
        [pltpu.VMEM((NCAM * CH,), jnp.int32) for _ in range(4 * NBUF)]
        + [pltpu.VMEM((CH,), jnp.int32) for _ in range(NCAM * NBUF)]
        + [pltpu.VMEM((CH, C), jnp.bfloat16) for _ in range(NBUF)]
        + [pltpu.VMEM((C, CH + 1), jnp.float32) for _ in range(NBUF)]
        + [pltpu.SemaphoreType.DMA for _ in range(3 * NBUF)]
    ),
    compiler_params=pltpu.CompilerParams(
        use_tc_tiling_on_sc=False, needs_layout_passes=False
    ),
)
def _fastray_sc(table, uu, vv, dd, va, out, *scratch):
    uub = scratch[0:NBUF]
    vvb = scratch[NBUF : 2 * NBUF]
    ddb = scratch[2 * NBUF : 3 * NBUF]
    vab = scratch[3 * NBUF : 4 * NBUF]
    k = 4 * NBUF
    idxb = [scratch[k + b * NCAM : k + (b + 1) * NCAM] for b in range(NBUF)]
    k += NCAM * NBUF
    accb = scratch[k : k + NBUF]
    acct = scratch[k + NBUF : k + 2 * NBUF]
    gsem = scratch[k + 2 * NBUF : k + 3 * NBUF]
    ssem = scratch[k + 3 * NBUF : k + 4 * NBUF]
    lsem = scratch[k + 4 * NBUF : k + 5 * NBUF]

    wid = lax.axis_index("s") * NC + lax.axis_index("c")
    base0 = jnp.where(wid == NW - 1, V - CH * CPW, wid * STRIDE)

    def super_body(i, carry):
        bases = [base0 + (i * NBUF + b) * CH for b in range(NBUF)]
        for b in range(NBUF):
            stage = []
            for n in range(NCAM):
                src = pl.ds(n * V + bases[b], CH)
                dst = pl.ds(n * CH, CH)
                for arr, buf in ((uu, uub), (vv, vvb), (dd, ddb), (va, vab)):
                    stage.append(
                        pltpu.async_copy(arr.at[src], buf[b].at[dst], lsem[b])
                    )
            for h in stage:
                h.wait()

            def grp(g, _, b=b):
                for n in range(NCAM):
                    off = pl.ds(n * CH + g * L, L)
                    lin = (
                        ddb[b][off] * (H * W)
                        + vvb[b][off] * W
                        + uub[b][off]
                        + n * DHW
                    )
                    idxb[b][n][pl.ds(g * L, L)] = lin | (vab[b][off] - 1)
                return 0

            lax.fori_loop(0, GROUPS, grp, 0)

            @pl.when(i > 0)
            def _():
                pltpu.make_async_copy(
                    acct[b].at[:, pl.ds(0, CH)], out.at[:, pl.ds(0, CH)], ssem[b]
                ).wait()

            def zrow(r, _, b=b):
                z = jnp.zeros((2 * L,), jnp.bfloat16)
                accb[b][r * 2, pl.ds(0, 2 * L)] = z
                accb[b][r * 2 + 1, pl.ds(0, 2 * L)] = z
                return 0

            lax.fori_loop(0, CH // 2, zrow, 0)

        handles = [[None] * NBUF for _ in range(NCAM)]
        for n in range(NCAM):
            for b in range(NBUF):
                if n > 0:
                    handles[n - 1][b].wait()
                handles[n][b] = pltpu.async_copy(
                    table.at[plsc.Indices(idxb[b][n], ignored_value=IGNORE)],
                    accb[b],
                    gsem[b],
                    add=True,
                )
        for b in range(NBUF):
            handles[NCAM - 1][b].wait()

            rows_lo = lax.iota(jnp.int32, 16) * 2
            rows_hi = rows_lo + 1

            def trow(r, _, b=b):
                cols = jnp.full((16,), r, jnp.int32)
                lo, hi = plsc.unpack(
                    accb[b][r, pl.ds(0, 2 * L)],
                    format=plsc.PackFormat.INTERLEAVED,
                )
                plsc.store_scatter(acct[b], [rows_lo, cols], lo)
                plsc.store_scatter(acct[b], [rows_hi, cols], hi)
                return 0

            lax.fori_loop(0, CH, trow, 0)
            pltpu.async_copy(
                acct[b].at[:, pl.ds(0, CH)],
                out.at[:, pl.ds(bases[b], CH)],
                ssem[b],
            )
        return carry

    lax.fori_loop(0, CPW // NBUF, super_body, 0)
    for b in range(NBUF):
        pltpu.make_async_copy(
            acct[b].at[:, pl.ds(0, CH)], out.at[:, pl.ds(0, CH)], ssem[b]
        ).wait()


def kernel(camera_features, uu, vv, dd, valid):
    B = camera_features.shape[0]
    feat = camera_features.astype(jnp.bfloat16).reshape(NCAM, C, DHW)
    table = jnp.swapaxes(feat, 1, 2).reshape(NCAM * DHW, C)
    out = _fastray_sc(
        table,
        uu.reshape(-1),
        vv.reshape(-1),
        dd.reshape(-1),
        valid.astype(jnp.int32).reshape(-1),
    )
    return out.reshape(B, C, *VOXEL_SHAPE)


# speedup vs baseline: 1.1904x; 1.1904x over previous
import functools

import jax
import jax.numpy as jnp
from jax import lax
from jax.experimental import pallas as pl
from jax.experimental.pallas import tpu as pltpu
from jax.experimental.pallas import tpu_sc as plsc

VOXEL_SHAPE = (4, 200, 200)
V = VOXEL_SHAPE[0] * VOXEL_SHAPE[1] * VOXEL_SHAPE[2]
NCAM = 6
C = 32
D, H, W = 32, 32, 60
DHW = D * H * W

NC, NS, L = 2, 16, 16
NW = NC * NS
CH = 512
CPW = 10
STRIDE = V // NW
NBUF = 2
GROUPS = CH // L
IGNORE = -1

_mesh = plsc.VectorSubcoreMesh(
    core_axis_name="c", subcore_axis_name="s", num_cores=NC, num_subcores=NS
)


@functools.partial(
    pl.kernel,
    out_type=jax.ShapeDtypeStruct((C, V), jnp.float32),
    mesh=_mesh,
    scratch_types=(
        [pltpu.VMEM((NCAM * CH,), jnp.int32) for _ in range(4 * NBUF)]
        + [pltpu.VMEM((CH,), jnp.int32) for _ in range(NCAM * NBUF)]
        + [pltpu.VMEM((CH, C), jnp.float32) for _ in range(NBUF)]
        + [pltpu.VMEM((C, CH + 1), jnp.float32) for _ in range(NBUF)]
        + [pltpu.SemaphoreType.DMA for _ in range(3 * NBUF)]
    ),
    compiler_params=pltpu.CompilerParams(
        use_tc_tiling_on_sc=False, needs_layout_passes=False
    ),
)
def _fastray_sc(table, uu, vv, dd, va, out, *scratch):
    uub = scratch[0:NBUF]
    vvb = scratch[NBUF : 2 * NBUF]
    ddb = scratch[2 * NBUF : 3 * NBUF]
    vab = scratch[3 * NBUF : 4 * NBUF]
    k = 4 * NBUF
    idxb = [scratch[k + b * NCAM : k + (b + 1) * NCAM] for b in range(NBUF)]
    k += NCAM * NBUF
    accb = scratch[k : k + NBUF]
    acct = scratch[k + NBUF : k + 2 * NBUF]
    gsem = scratch[k + 2 * NBUF : k + 3 * NBUF]
    ssem = scratch[k + 3 * NBUF : k + 4 * NBUF]
    lsem = scratch[k + 4 * NBUF : k + 5 * NBUF]

    wid = lax.axis_index("s") * NC + lax.axis_index("c")
    base0 = jnp.where(wid == NW - 1, V - CH * CPW, wid * STRIDE)

    def super_body(i, carry):
        bases = [base0 + (i * NBUF + b) * CH for b in range(NBUF)]
        for b in range(NBUF):
            stage = []
            for n in range(NCAM):
                src = pl.ds(n * V + bases[b], CH)
                dst = pl.ds(n * CH, CH)
                for arr, buf in ((uu, uub), (vv, vvb), (dd, ddb), (va, vab)):
                    stage.append(
                        pltpu.async_copy(arr.at[src], buf[b].at[dst], lsem[b])
                    )
            for h in stage:
                h.wait()

            def grp(g, _, b=b):
                for n in range(NCAM):
                    off = pl.ds(n * CH + g * L, L)
                    lin = (
                        ddb[b][off] * (H * W)
                        + vvb[b][off] * W
                        + uub[b][off]
                        + n * DHW
                    )
                    idxb[b][n][pl.ds(g * L, L)] = lin | (vab[b][off] - 1)
                return 0

            lax.fori_loop(0, GROUPS, grp, 0)

            @pl.when(i > 0)
            def _():
                pltpu.make_async_copy(
                    acct[b].at[:, pl.ds(0, CH)], out.at[:, pl.ds(0, CH)], ssem[b]
                ).wait()

            def zrow(r, _, b=b):
                z = jnp.zeros((L,), jnp.float32)
                accb[b][r * 2, pl.ds(0, L)] = z
                accb[b][r * 2, pl.ds(L, L)] = z
                accb[b][r * 2 + 1, pl.ds(0, L)] = z
                accb[b][r * 2 + 1, pl.ds(L, L)] = z
                return 0

            lax.fori_loop(0, CH // 2, zrow, 0)

        handles = [[None] * NBUF for _ in range(NCAM)]
        for n in range(NCAM):
            for b in range(NBUF):
                handles[n][b] = pltpu.async_copy(
                    table.at[plsc.Indices(idxb[b][n], ignored_value=IGNORE)],
                    accb[b],
                    gsem[b],
                    add=True,
                )
        for b in range(NBUF):
            for n in range(NCAM):
                handles[n][b].wait()

            rows_lo = lax.iota(jnp.int32, 16)
            rows_hi = rows_lo + 16

            def trow(r, _, b=b):
                cols = jnp.full((16,), r, jnp.int32)
                plsc.store_scatter(
                    acct[b], [rows_lo, cols], accb[b][r, pl.ds(0, L)]
                )
                plsc.store_scatter(
                    acct[b], [rows_hi, cols], accb[b][r, pl.ds(L, L)]
                )
                return 0

            lax.fori_loop(0, CH, trow, 0)
            pltpu.async_copy(
                acct[b].at[:, pl.ds(0, CH)],
                out.at[:, pl.ds(bases[b], CH)],
                ssem[b],
            )
        return carry

    lax.fori_loop(0, CPW // NBUF, super_body, 0)
    for b in range(NBUF):
        pltpu.make_async_copy(
            acct[b].at[:, pl.ds(0, CH)], out.at[:, pl.ds(0, CH)], ssem[b]
        ).wait()


def kernel(camera_features, uu, vv, dd, valid):
    B = camera_features.shape[0]
    feat = camera_features.reshape(NCAM, C, DHW)
    table = jnp.swapaxes(feat, 1, 2).reshape(NCAM * DHW, C)
    out = _fastray_sc(
        table,
        uu.reshape(-1),
        vv.reshape(-1),
        dd.reshape(-1),
        valid.astype(jnp.int32).reshape(-1),
    )
    return out.reshape(B, C, *VOXEL_SHAPE)


# speedup vs baseline: 1.2057x; 1.0129x over previous
import functools

import jax
import jax.numpy as jnp
from jax import lax
from jax.experimental import pallas as pl
from jax.experimental.pallas import tpu as pltpu
from jax.experimental.pallas import tpu_sc as plsc

VOXEL_SHAPE = (4, 200, 200)
V = VOXEL_SHAPE[0] * VOXEL_SHAPE[1] * VOXEL_SHAPE[2]
NCAM = 6
C = 32
D, H, W = 32, 32, 60
DHW = D * H * W

NC, NS, L = 2, 16, 16
NW = NC * NS
CH = 512
CPW = 10
STRIDE = V // NW
NBUF = 2
GROUPS = CH // L
IGNORE = -1

_mesh = plsc.VectorSubcoreMesh(
    core_axis_name="c", subcore_axis_name="s", num_cores=NC, num_subcores=NS
)


@functools.partial(
    pl.kernel,
    out_type=jax.ShapeDtypeStruct((C, V), jnp.float32),
    mesh=_mesh,
    scratch_types=(
        [pltpu.VMEM((NCAM * CH,), jnp.int32) for _ in range(4 * NBUF)]
        + [pltpu.VMEM((CH,), jnp.int32) for _ in range(NCAM * NBUF)]
        + [pltpu.VMEM((CH, C), jnp.float32) for _ in range(NBUF)]
        + [pltpu.VMEM((C, CH + 1), jnp.float32) for _ in range(NBUF)]
        + [pltpu.SemaphoreType.DMA for _ in range(3 * NBUF)]
    ),
    compiler_params=pltpu.CompilerParams(
        use_tc_tiling_on_sc=False, needs_layout_passes=False
    ),
)
def _fastray_sc(table, uu, vv, dd, va, out, *scratch):
    uub = scratch[0:NBUF]
    vvb = scratch[NBUF : 2 * NBUF]
    ddb = scratch[2 * NBUF : 3 * NBUF]
    vab = scratch[3 * NBUF : 4 * NBUF]
    k = 4 * NBUF
    idxb = [scratch[k + b * NCAM : k + (b + 1) * NCAM] for b in range(NBUF)]
    k += NCAM * NBUF
    accb = scratch[k : k + NBUF]
    acct = scratch[k + NBUF : k + 2 * NBUF]
    gsem = scratch[k + 2 * NBUF : k + 3 * NBUF]
    ssem = scratch[k + 3 * NBUF : k + 4 * NBUF]
    lsem = scratch[k + 4 * NBUF : k + 5 * NBUF]

    wid = lax.axis_index("s") * NC + lax.axis_index("c")
    base0 = jnp.where(wid == NW - 1, V - CH * CPW, wid * STRIDE)

    rows_lo = lax.iota(jnp.int32, 16)
    rows_hi = rows_lo + 16

    def drain_and_store(i, b, store_pred=None):
        base = base0 + (i * NBUF + b) * CH
        for n in range(NCAM):
            pltpu.make_async_copy(
                table.at[plsc.Indices(idxb[b][n], ignored_value=IGNORE)],
                accb[b],
                gsem[b],
            ).wait()

        def _wait_prev_store():
            pltpu.make_async_copy(
                acct[b].at[:, pl.ds(0, CH)], out.at[:, pl.ds(0, CH)], ssem[b]
            ).wait()

        if store_pred is None:
            _wait_prev_store()
        else:
            pl.when(store_pred)(_wait_prev_store)

        def trow(r, _, b=b):
            cols = jnp.full((16,), r, jnp.int32)
            plsc.store_scatter(acct[b], [rows_lo, cols], accb[b][r, pl.ds(0, L)])
            plsc.store_scatter(acct[b], [rows_hi, cols], accb[b][r, pl.ds(L, L)])
            return 0

        lax.fori_loop(0, CH, trow, 0)
        pltpu.async_copy(
            acct[b].at[:, pl.ds(0, CH)], out.at[:, pl.ds(base, CH)], ssem[b]
        )

    def super_body(i, carry):
        bases = [base0 + (i * NBUF + b) * CH for b in range(NBUF)]
        stage = []
        for b in range(NBUF):
            for n in range(NCAM):
                src = pl.ds(n * V + bases[b], CH)
                dst = pl.ds(n * CH, CH)
                for arr, buf in ((uu, uub), (vv, vvb), (dd, ddb), (va, vab)):
                    stage.append(
                        pltpu.async_copy(arr.at[src], buf[b].at[dst], lsem[b])
                    )

        @pl.when(i > 0)
        def _():
            for b in range(NBUF):
                drain_and_store(i - 1, b, store_pred=i > 1)

        for h in stage:
            h.wait()
        for b in range(NBUF):

            def grp(g, _, b=b):
                for n in range(NCAM):
                    off = pl.ds(n * CH + g * L, L)
                    lin = (
                        ddb[b][off] * (H * W)
                        + vvb[b][off] * W
                        + uub[b][off]
                        + n * DHW
                    )
                    idxb[b][n][pl.ds(g * L, L)] = lin | (vab[b][off] - 1)
                return 0

            lax.fori_loop(0, GROUPS, grp, 0)

            def zrow(r, _, b=b):
                z = jnp.zeros((L,), jnp.float32)
                accb[b][r * 2, pl.ds(0, L)] = z
                accb[b][r * 2, pl.ds(L, L)] = z
                accb[b][r * 2 + 1, pl.ds(0, L)] = z
                accb[b][r * 2 + 1, pl.ds(L, L)] = z
                return 0

            lax.fori_loop(0, CH // 2, zrow, 0)

        for n in range(NCAM):
            for b in range(NBUF):
                pltpu.async_copy(
                    table.at[plsc.Indices(idxb[b][n], ignored_value=IGNORE)],
                    accb[b],
                    gsem[b],
                    add=True,
                )
        return carry

    steps = CPW // NBUF
    lax.fori_loop(0, steps, super_body, 0)
    for b in range(NBUF):
        drain_and_store(steps - 1, b)
    for b in range(NBUF):
        pltpu.make_async_copy(
            acct[b].at[:, pl.ds(0, CH)], out.at[:, pl.ds(0, CH)], ssem[b]
        ).wait()


def kernel(camera_features, uu, vv, dd, valid):
    B = camera_features.shape[0]
    feat = camera_features.reshape(NCAM, C, DHW)
    table = jnp.swapaxes(feat, 1, 2).reshape(NCAM * DHW, C)
    out = _fastray_sc(
        table,
        uu.reshape(-1),
        vv.reshape(-1),
        dd.reshape(-1),
        valid.astype(jnp.int32).reshape(-1),
    )
    return out.reshape(B, C, *VOXEL_SHAPE)
